# baseline (speedup 1.0000x reference)
import jax
import jax.numpy as jnp
from jax import lax
from jax.experimental import pallas as pl
from jax.experimental.pallas import tpu as pltpu
from jax.experimental.pallas import tpu_sc as plsc

_E = 16
_B = 16384
_S = 21
_NC, _NS, _L = 2, 16, 16
_NW = _NC * _NS
_BW = _B // _NW
_C = 128
_NCH = _BW // _C
_CS = _C * _S
_SB = _L


def _sc_body(tgt_hbm, oidx_hbm, in_hbm, out_hbm, scores_hbm,
             tgt_idx, oidx, t_rows, o_rows, scores, sem):
  cid = lax.axis_index("c")
  sid = lax.axis_index("s")
  wid = sid * _NC + cid
  lanes = lax.iota(jnp.int32, _L)
  e_idx = [jnp.full((_L,), e, jnp.int32) for e in range(_E)]

  for ch in range(_NCH):
    g = wid * _NCH + ch
    pltpu.sync_copy(tgt_hbm.at[g], tgt_idx)
    pltpu.sync_copy(oidx_hbm.at[g], oidx)
    cps = [pltpu.async_copy(in_hbm.at[tgt_idx.at[0]], t_rows, sem)]
    for j in range(_S):
      cps.append(pltpu.async_copy(out_hbm.at[oidx.at[j]],
                                  o_rows.at[pl.ds(j * _C, _C)], sem))
    for cp in cps:
      cp.wait()

    def block(sb, carry):
      s_loc = sb * _SB + lanes
      t_cols = [plsc.load_gather(t_rows, [s_loc, e_idx[e]]) for e in range(_E)]
      s21 = s_loc * _S
      for j in range(_S):
        kk = s21 + j
        acc = t_cols[0] * plsc.load_gather(o_rows, [kk, e_idx[0]])
        for e in range(1, _E):
          acc = acc + t_cols[e] * plsc.load_gather(o_rows, [kk, e_idx[e]])
        if j > 0:
          acc = -acc
        plsc.store_scatter(scores, [kk >> 7, kk & 127], acc)
      return carry

    lax.fori_loop(0, _C // _SB, block, 0)
    pltpu.sync_copy(scores, scores_hbm.at[g])


_NG = _NW * _NCH

_sc_scores = pl.kernel(
    _sc_body,
    out_type=jax.ShapeDtypeStruct((_NG, _S, 128), jnp.float32),
    mesh=plsc.VectorSubcoreMesh(core_axis_name="c", subcore_axis_name="s"),
    compiler_params=pltpu.CompilerParams(
        needs_layout_passes=False, use_tc_tiling_on_sc=False),
    scratch_types=[
        pltpu.VMEM((1, 128), jnp.int32),
        pltpu.VMEM((_S, 128), jnp.int32),
        pltpu.VMEM((_C, _E), jnp.float32),
        pltpu.VMEM((_CS, _E), jnp.float32),
        pltpu.VMEM((_S, 128), jnp.float32),
        pltpu.SemaphoreType.DMA,
    ],
)


def _tc_body(scores_ref, out_ref):
  x = scores_ref[...]
  m = jnp.maximum(x, 0.0)
  ls = x - m - jnp.log(jnp.exp(-m) + jnp.exp(x - m))
  out_ref[...] = (-jnp.sum(ls) * (1.0 / _B))[None, None]


_tc_loss = pl.pallas_call(
    _tc_body,
    out_shape=jax.ShapeDtypeStruct((1, 1), jnp.float32),
)


def kernel(target, context, noise_words, in_table, out_table):
  tgt3d = target.astype(jnp.int32).reshape(_NG, 1, 128)
  oidx3d = jnp.concatenate(
      [context[:, None], noise_words], axis=1).astype(jnp.int32).reshape(
          _NG, _S, 128)
  scores = _sc_scores(tgt3d, oidx3d, in_table, out_table)
  loss = _tc_loss(scores.reshape(_B * _S // 128, 128))
  return loss[0, 0]
